# baseline (speedup 1.0000x reference)
import functools

import jax
import jax.numpy as jnp
from jax import lax
from jax.experimental import pallas as pl
from jax.experimental.pallas import tpu as pltpu
from jax.experimental.pallas import tpu_sc as plsc

M = 128
N = 65472
NPAD = 65536
NUM_CLASSES = 8
K = 9
IMG = 1024.0
LEVEL_WIDTHS = [49152, 12288, 3072, 768, 192]
LEVEL_STARTS = [0, 49152, 61440, 64512, 65280]
LEVEL_CHUNKS = [(3, 16384), (1, 12288), (1, 3072), (1, 768), (1, 192)]
STRIDES = [8, 16, 32, 64, 128]
LOG2_G2 = [14, 12, 10, 8, 6]
LOG2_GRID = [7, 6, 5, 4, 3]
BIG = 2 ** 30
CAND = 45
CANDP = 48
ROW_CHUNKS = [(0, 16384), (16384, 16384), (32768, 16384), (49152, 16320)]


def _topk_body(level_start, cw, single, x_ref, ov_ref, oi_ref):
    i = pl.program_id(0)

    if not single:
        @pl.when(i == 0)
        def _():
            ov_ref[...] = jnp.full((M, 16), -1.0, jnp.float32)
            oi_ref[...] = jnp.zeros((M, 16), jnp.int32)

    x = x_ref[...]
    liota = lax.broadcasted_iota(jnp.int32, (M, cw), 1)
    base = level_start + i * cw
    lane16 = lax.broadcasted_iota(jnp.int32, (M, 16), 1)

    cv = jnp.full((M, 16), -1.0, jnp.float32)
    ci = jnp.zeros((M, 16), jnp.int32)
    for t in range(K):
        m = jnp.max(x, axis=1, keepdims=True)
        idxl = jnp.argmax(x, axis=1, keepdims=True).astype(jnp.int32)
        cv = jnp.where(lane16 == t, m, cv)
        ci = jnp.where(lane16 == t, idxl + base, ci)
        x = jnp.where(liota == idxl, -1.0, x)

    if single:
        ov_ref[...] = cv
        oi_ref[...] = ci
        return

    pv = jnp.concatenate([ov_ref[...], cv], axis=1)
    pi = jnp.concatenate([oi_ref[...], ci], axis=1)
    nv = jnp.full((M, 16), -1.0, jnp.float32)
    ni = jnp.zeros((M, 16), jnp.int32)
    for t in range(K):
        m = jnp.max(pv, axis=1, keepdims=True)
        sel = pv == m
        idx = jnp.min(jnp.where(sel, pi, BIG), axis=1, keepdims=True)
        nv = jnp.where(lane16 == t, m, nv)
        ni = jnp.where(lane16 == t, idx, ni)
        pv = jnp.where(sel & (pi == idx), -2.0, pv)
    ov_ref[...] = nv
    oi_ref[...] = ni


def _level_topk(x, level_start, n_chunks, cw):
    return pl.pallas_call(
        functools.partial(_topk_body, level_start, cw, n_chunks == 1),
        grid=(n_chunks,),
        in_specs=[pl.BlockSpec((M, cw), lambda i: (0, i))],
        out_specs=[pl.BlockSpec((M, 16), lambda i: (0, 0)),
                   pl.BlockSpec((M, 16), lambda i: (0, 0))],
        out_shape=[jax.ShapeDtypeStruct((M, 16), jnp.float32),
                   jax.ShapeDtypeStruct((M, 16), jnp.int32)],
    )(x)


def _splat(vec, i):
    return jnp.take(vec, jnp.full((16,), i, jnp.int32))


def _decode_anchor(c):
    ax0 = jnp.zeros((16,), jnp.float32)
    ay0 = jnp.zeros((16,), jnp.float32)
    ax1 = jnp.zeros((16,), jnp.float32)
    ay1 = jnp.zeros((16,), jnp.float32)
    for lvl in range(5):
        start = LEVEL_STARTS[lvl]
        stride = float(STRIDES[lvl])
        r = c - start
        scale = lax.shift_right_logical(r, LOG2_G2[lvl])
        p = jnp.bitwise_and(r, (1 << LOG2_G2[lvl]) - 1)
        row_g = lax.shift_right_logical(p, LOG2_GRID[lvl])
        col_g = jnp.bitwise_and(p, (1 << LOG2_GRID[lvl]) - 1)
        cx = (col_g.astype(jnp.float32) + 0.5) * stride
        cy = (row_g.astype(jnp.float32) + 0.5) * stride
        h = (scale.astype(jnp.float32) * 0.5 + 2.0) * stride
        inl = (c >= start) & (c < start + LEVEL_WIDTHS[lvl])
        ax0 = jnp.where(inl, cx - h, ax0)
        ay0 = jnp.where(inl, cy - h, ay0)
        ax1 = jnp.where(inl, cx + h, ax1)
        ay1 = jnp.where(inl, cy + h, ay1)
    return ax0, ay0, ax1, ay1


def _sc_body(cent_ref, cidx_ref, cval_ref, tgt_ref, mod_ref, ign_ref,
             buf, civ, cvv, tv, ignbuf):
    w = lax.axis_index("s") * 2 + lax.axis_index("c")
    lane = lax.iota(jnp.int32, 16)
    pltpu.sync_copy(tgt_ref.at[pl.ds(w * 16, 16)], tv)
    tvec = tv[...]

    cols_per_sub = NPAD // 32

    def _ign_iter(i, carry):
        c = lane + (w * cols_per_sub + i * 16)
        ax0, ay0, ax1, ay1 = _decode_anchor(c)
        bad = (ax0 < 0.0) | (ay0 < 0.0) | (ax1 > IMG) | (ay1 > IMG)
        ignbuf[pl.ds(i * 16, 16)] = jnp.where(bad, jnp.float32(1.0),
                                              jnp.float32(0.0))
        return carry

    lax.fori_loop(0, cols_per_sub // 16, _ign_iter, 0)
    pltpu.sync_copy(ignbuf, ign_ref.at[pl.ds(w * cols_per_sub,
                                             cols_per_sub)])

    for j in range(4):
        r = w * 4 + j
        pltpu.sync_copy(cidx_ref.at[pl.ds(r * CANDP, CANDP)], civ)
        pltpu.sync_copy(cval_ref.at[pl.ds(r * CANDP, CANDP)], cvv)
        gx0 = _splat(tvec, 4 * j + 0)
        gy0 = _splat(tvec, 4 * j + 1)
        gx1 = _splat(tvec, 4 * j + 2)
        gy1 = _splat(tvec, 4 * j + 3)
        area_g = (gx1 - gx0) * (gy1 - gy0)

        cs, vals, valids, ious = [], [], [], []
        for k in range(3):
            c = civ[pl.ds(16 * k, 16)]
            v = cvv[pl.ds(16 * k, 16)]
            valid = (lane + 16 * k) < CAND
            ax0, ay0, ax1, ay1 = _decode_anchor(c)
            ltx = jnp.maximum(gx0, ax0)
            lty = jnp.maximum(gy0, ay0)
            rbx = jnp.minimum(gx1, ax1)
            rby = jnp.minimum(gy1, ay1)
            iw = jnp.maximum(rbx - ltx, 0.0)
            ih = jnp.maximum(rby - lty, 0.0)
            inter = iw * ih
            area_a = (ax1 - ax0) * (ay1 - ay0)
            union = area_g + area_a - inter
            iou = inter / jnp.maximum(union, 1e-9)
            cs.append(c)
            vals.append(v)
            valids.append(valid)
            ious.append(iou)

        s = jnp.float32(0.0)
        for k in range(3):
            s = s + lax.reduce_sum_p.bind(
                jnp.where(valids[k], ious[k], 0.0), axes=(0,))
        mean = jnp.full((16,), s, jnp.float32) / jnp.float32(CAND)
        ss = jnp.float32(0.0)
        ds = []
        for k in range(3):
            d = ious[k] - mean
            ds.append(d)
            ss = ss + lax.reduce_sum_p.bind(
                jnp.where(valids[k], d * d, 0.0), axes=(0,))
        var = jnp.full((16,), ss, jnp.float32) / jnp.float32(CAND - 1)

        scat_vals, scat_masks = [], []
        for k in range(3):
            pos = (ds[k] > 0.0) & (ds[k] * ds[k] > var) & (vals[k] > 0.0)
            scat_vals.append(jnp.where(pos, jnp.float32(1.1),
                                       jnp.float32(0.0)))
            scat_masks.append(valids[k])

        for (st, sz) in ROW_CHUNKS:
            pltpu.sync_copy(cent_ref.at[pl.ds(r * N + st, sz)],
                            buf.at[pl.ds(0, sz)])
            for k in range(3):
                inchunk = scat_masks[k] & (cs[k] >= st) & (cs[k] < st + sz)
                loc = jnp.clip(cs[k] - st, 0, sz - 1)
                plsc.store_scatter(buf, [loc], scat_vals[k], mask=inchunk)
            pltpu.sync_copy(buf.at[pl.ds(0, sz)],
                            mod_ref.at[pl.ds(r * NPAD + st, sz)])


def _sc_scatter(cent_flat, cidx_flat, cval_flat, tgt_flat):
    mesh = plsc.VectorSubcoreMesh(core_axis_name="c", subcore_axis_name="s")
    return pl.kernel(
        _sc_body,
        out_type=[jax.ShapeDtypeStruct((M * NPAD,), jnp.float32),
                  jax.ShapeDtypeStruct((NPAD,), jnp.float32)],
        mesh=mesh,
        compiler_params=pltpu.CompilerParams(needs_layout_passes=False),
        scratch_types=[
            pltpu.VMEM((16384,), jnp.float32),
            pltpu.VMEM((CANDP,), jnp.int32),
            pltpu.VMEM((CANDP,), jnp.float32),
            pltpu.VMEM((16,), jnp.float32),
            pltpu.VMEM((NPAD // 32,), jnp.float32),
        ],
    )(cent_flat, cidx_flat, cval_flat, tgt_flat)


CB = 4096


def _assign_body(mod_ref, ign_ref, oh_ref, mt_ref, lb_ref):
    x = mod_ref[...]
    ign = ign_ref[...][0] > 0.0
    neginf = jnp.float32(-jnp.inf)
    mrows, lrows = [], []
    for c in range(NUM_CLASSES):
        mc = oh_ref[:, c:c + 1] > 0.0
        hasany = jnp.max(oh_ref[:, c:c + 1]) > 0.0
        xm = jnp.where(mc, x, neginf)
        mx = jnp.max(xm, axis=0, keepdims=True)
        am = jnp.argmax(xm, axis=0, keepdims=True).astype(jnp.int32)
        pos = mx == jnp.float32(1.1)
        lab = jnp.where(pos, jnp.int32(1), jnp.int32(0))
        lab = jnp.where(ign, jnp.int32(-1), lab)
        mrows.append(jnp.where(hasany, am, 0))
        lrows.append(jnp.where(hasany, lab, 0))
    mt_ref[...] = jnp.concatenate(mrows, axis=0)
    lb_ref[...] = jnp.concatenate(lrows, axis=0)


def _assign(mod, ign3, onehot):
    return pl.pallas_call(
        _assign_body,
        grid=(NPAD // CB,),
        in_specs=[pl.BlockSpec((M, CB), lambda i: (0, i)),
                  pl.BlockSpec((1, 1, CB), lambda i: (i, 0, 0)),
                  pl.BlockSpec((M, NUM_CLASSES), lambda i: (0, 0))],
        out_specs=[pl.BlockSpec((NUM_CLASSES, CB), lambda i: (0, i)),
                   pl.BlockSpec((NUM_CLASSES, CB), lambda i: (0, i))],
        out_shape=[jax.ShapeDtypeStruct((NUM_CLASSES, NPAD), jnp.int32),
                   jax.ShapeDtypeStruct((NUM_CLASSES, NPAD), jnp.int32)],
    )(mod, ign3, onehot)


def kernel(centerness_matrix, targets, anchors, gt_labels):
    cvs, cis = [], []
    for lvl in range(5):
        start = LEVEL_STARTS[lvl]
        n_chunks, cw = LEVEL_CHUNKS[lvl]
        cv, ci = _level_topk(
            centerness_matrix[:, start:start + LEVEL_WIDTHS[lvl]],
            start, n_chunks, cw)
        cvs.append(cv[:, :K])
        cis.append(ci[:, :K])
    cand_val = jnp.concatenate(cvs, axis=1)
    cand_idx = jnp.concatenate(cis, axis=1)
    cand_val = jnp.pad(cand_val, ((0, 0), (0, CANDP - CAND)),
                       constant_values=-1.0)
    cand_idx = jnp.pad(cand_idx, ((0, 0), (0, CANDP - CAND)))

    mod_flat, ign_flat = _sc_scatter(
        centerness_matrix.reshape(-1),
        cand_idx.reshape(-1),
        cand_val.reshape(-1),
        targets.reshape(-1),
    )
    mod = mod_flat.reshape(M, NPAD)
    ign3 = ign_flat.reshape(NPAD // CB, 1, CB)

    onehot = (gt_labels[:, None] ==
              jnp.arange(NUM_CLASSES)[None, :]).astype(jnp.float32)
    mt, lb = _assign(mod, ign3, onehot)
    matches = mt[:, :N].T
    match_labels = lb[:, :N].T.astype(jnp.int8)
    return matches, match_labels


# speedup vs baseline: 1.0150x; 1.0150x over previous
import functools

import jax
import jax.numpy as jnp
from jax import lax
from jax.experimental import pallas as pl
from jax.experimental.pallas import tpu as pltpu
from jax.experimental.pallas import tpu_sc as plsc

M = 128
N = 65472
NPAD = 65536
NUM_CLASSES = 8
K = 9
IMG = 1024.0
LEVEL_WIDTHS = [49152, 12288, 3072, 768, 192]
LEVEL_STARTS = [0, 49152, 61440, 64512, 65280]
LEVEL_CHUNKS = [(3, 16384), (1, 12288), (1, 3072), (1, 768), (1, 192)]
STRIDES = [8, 16, 32, 64, 128]
LOG2_G2 = [14, 12, 10, 8, 6]
LOG2_GRID = [7, 6, 5, 4, 3]
BIG = 2 ** 30
CAND = 45
CANDP = 48
ROW_CHUNKS = [(0, 16384), (16384, 16384), (32768, 16384), (49152, 16320)]


def _topk_body(level_start, cw, single, x_ref, ov_ref, oi_ref):
    i = pl.program_id(0)

    if not single:
        @pl.when(i == 0)
        def _():
            ov_ref[...] = jnp.full((M, 16), -1.0, jnp.float32)
            oi_ref[...] = jnp.zeros((M, 16), jnp.int32)

    x = x_ref[...]
    liota = lax.broadcasted_iota(jnp.int32, (M, cw), 1)
    base = level_start + i * cw
    lane16 = lax.broadcasted_iota(jnp.int32, (M, 16), 1)

    cv = jnp.full((M, 16), -1.0, jnp.float32)
    ci = jnp.zeros((M, 16), jnp.int32)
    for t in range(K):
        m = jnp.max(x, axis=1, keepdims=True)
        idxl = jnp.argmax(x, axis=1, keepdims=True).astype(jnp.int32)
        cv = jnp.where(lane16 == t, m, cv)
        ci = jnp.where(lane16 == t, idxl + base, ci)
        x = jnp.where(liota == idxl, -1.0, x)

    if single:
        ov_ref[...] = cv
        oi_ref[...] = ci
        return

    pv = jnp.concatenate([ov_ref[...], cv], axis=1)
    pi = jnp.concatenate([oi_ref[...], ci], axis=1)
    nv = jnp.full((M, 16), -1.0, jnp.float32)
    ni = jnp.zeros((M, 16), jnp.int32)
    for t in range(K):
        m = jnp.max(pv, axis=1, keepdims=True)
        sel = pv == m
        idx = jnp.min(jnp.where(sel, pi, BIG), axis=1, keepdims=True)
        nv = jnp.where(lane16 == t, m, nv)
        ni = jnp.where(lane16 == t, idx, ni)
        pv = jnp.where(sel & (pi == idx), -2.0, pv)
    ov_ref[...] = nv
    oi_ref[...] = ni


def _level_topk(x, level_start, n_chunks, cw):
    return pl.pallas_call(
        functools.partial(_topk_body, level_start, cw, n_chunks == 1),
        grid=(n_chunks,),
        in_specs=[pl.BlockSpec((M, cw), lambda i: (0, i))],
        out_specs=[pl.BlockSpec((M, 16), lambda i: (0, 0)),
                   pl.BlockSpec((M, 16), lambda i: (0, 0))],
        out_shape=[jax.ShapeDtypeStruct((M, 16), jnp.float32),
                   jax.ShapeDtypeStruct((M, 16), jnp.int32)],
    )(x)


def _splat(vec, i):
    return jnp.take(vec, jnp.full((16,), i, jnp.int32))


def _decode_anchor(c):
    ax0 = jnp.zeros((16,), jnp.float32)
    ay0 = jnp.zeros((16,), jnp.float32)
    ax1 = jnp.zeros((16,), jnp.float32)
    ay1 = jnp.zeros((16,), jnp.float32)
    for lvl in range(5):
        start = LEVEL_STARTS[lvl]
        stride = float(STRIDES[lvl])
        r = c - start
        scale = lax.shift_right_logical(r, LOG2_G2[lvl])
        p = jnp.bitwise_and(r, (1 << LOG2_G2[lvl]) - 1)
        row_g = lax.shift_right_logical(p, LOG2_GRID[lvl])
        col_g = jnp.bitwise_and(p, (1 << LOG2_GRID[lvl]) - 1)
        cx = (col_g.astype(jnp.float32) + 0.5) * stride
        cy = (row_g.astype(jnp.float32) + 0.5) * stride
        h = (scale.astype(jnp.float32) * 0.5 + 2.0) * stride
        inl = (c >= start) & (c < start + LEVEL_WIDTHS[lvl])
        ax0 = jnp.where(inl, cx - h, ax0)
        ay0 = jnp.where(inl, cy - h, ay0)
        ax1 = jnp.where(inl, cx + h, ax1)
        ay1 = jnp.where(inl, cy + h, ay1)
    return ax0, ay0, ax1, ay1


def _sc_body(cent_ref, cidx_ref, cval_ref, tgt_ref, mod_ref, ign_ref,
             buf, civ, cvv, tv, ignbuf):
    w = lax.axis_index("s") * 2 + lax.axis_index("c")
    lane = lax.iota(jnp.int32, 16)
    pltpu.sync_copy(tgt_ref.at[pl.ds(w * 16, 16)], tv)
    tvec = tv[...]

    cols_per_sub = NPAD // 32

    def _ign_iter(i, carry):
        c = lane + (w * cols_per_sub + i * 16)
        ax0, ay0, ax1, ay1 = _decode_anchor(c)
        bad = (ax0 < 0.0) | (ay0 < 0.0) | (ax1 > IMG) | (ay1 > IMG)
        ignbuf[pl.ds(i * 16, 16)] = jnp.where(bad, jnp.float32(1.0),
                                              jnp.float32(0.0))
        return carry

    lax.fori_loop(0, cols_per_sub // 16, _ign_iter, 0)
    pltpu.sync_copy(ignbuf, ign_ref.at[pl.ds(w * cols_per_sub,
                                             cols_per_sub)])

    for j in range(4):
        r = w * 4 + j
        pltpu.sync_copy(cidx_ref.at[pl.ds(r * CANDP, CANDP)], civ)
        pltpu.sync_copy(cval_ref.at[pl.ds(r * CANDP, CANDP)], cvv)
        gx0 = _splat(tvec, 4 * j + 0)
        gy0 = _splat(tvec, 4 * j + 1)
        gx1 = _splat(tvec, 4 * j + 2)
        gy1 = _splat(tvec, 4 * j + 3)
        area_g = (gx1 - gx0) * (gy1 - gy0)

        cs, vals, valids, ious = [], [], [], []
        for k in range(3):
            c = civ[pl.ds(16 * k, 16)]
            v = cvv[pl.ds(16 * k, 16)]
            valid = (lane + 16 * k) < CAND
            ax0, ay0, ax1, ay1 = _decode_anchor(c)
            ltx = jnp.maximum(gx0, ax0)
            lty = jnp.maximum(gy0, ay0)
            rbx = jnp.minimum(gx1, ax1)
            rby = jnp.minimum(gy1, ay1)
            iw = jnp.maximum(rbx - ltx, 0.0)
            ih = jnp.maximum(rby - lty, 0.0)
            inter = iw * ih
            area_a = (ax1 - ax0) * (ay1 - ay0)
            union = area_g + area_a - inter
            iou = inter / jnp.maximum(union, 1e-9)
            cs.append(c)
            vals.append(v)
            valids.append(valid)
            ious.append(iou)

        s = jnp.float32(0.0)
        for k in range(3):
            s = s + lax.reduce_sum_p.bind(
                jnp.where(valids[k], ious[k], 0.0), axes=(0,))
        mean = jnp.full((16,), s, jnp.float32) / jnp.float32(CAND)
        ss = jnp.float32(0.0)
        ds = []
        for k in range(3):
            d = ious[k] - mean
            ds.append(d)
            ss = ss + lax.reduce_sum_p.bind(
                jnp.where(valids[k], d * d, 0.0), axes=(0,))
        var = jnp.full((16,), ss, jnp.float32) / jnp.float32(CAND - 1)

        scat_vals, scat_masks = [], []
        for k in range(3):
            pos = (ds[k] > 0.0) & (ds[k] * ds[k] > var) & (vals[k] > 0.0)
            scat_vals.append(jnp.where(pos, jnp.float32(1.1),
                                       jnp.float32(0.0)))
            scat_masks.append(valids[k])

        for (st, sz) in ROW_CHUNKS:
            pltpu.sync_copy(cent_ref.at[pl.ds(r * N + st, sz)],
                            buf.at[pl.ds(0, sz)])
            for k in range(3):
                inchunk = scat_masks[k] & (cs[k] >= st) & (cs[k] < st + sz)
                loc = jnp.clip(cs[k] - st, 0, sz - 1)
                plsc.store_scatter(buf, [loc], scat_vals[k], mask=inchunk)
            pltpu.sync_copy(buf.at[pl.ds(0, sz)],
                            mod_ref.at[pl.ds(r * NPAD + st, sz)])


def _sc_scatter(cent_flat, cidx_flat, cval_flat, tgt_flat):
    mesh = plsc.VectorSubcoreMesh(core_axis_name="c", subcore_axis_name="s")
    return pl.kernel(
        _sc_body,
        out_type=[jax.ShapeDtypeStruct((M * NPAD,), jnp.float32),
                  jax.ShapeDtypeStruct((NPAD,), jnp.float32)],
        mesh=mesh,
        compiler_params=pltpu.CompilerParams(needs_layout_passes=False),
        scratch_types=[
            pltpu.VMEM((16384,), jnp.float32),
            pltpu.VMEM((CANDP,), jnp.int32),
            pltpu.VMEM((CANDP,), jnp.float32),
            pltpu.VMEM((16,), jnp.float32),
            pltpu.VMEM((NPAD // 32,), jnp.float32),
        ],
    )(cent_flat, cidx_flat, cval_flat, tgt_flat)


CB = 2048


def _assign_body(mod_ref, ign_ref, oh_ref, mt_ref, lb_ref):
    x = mod_ref[...]
    ign = ign_ref[...][0] > 0.0
    neginf = jnp.float32(-jnp.inf)
    mrows, lrows = [], []
    for c in range(NUM_CLASSES):
        mc = oh_ref[:, c:c + 1] > 0.0
        hasany = jnp.max(oh_ref[:, c:c + 1]) > 0.0
        xm = jnp.where(mc, x, neginf)
        mx = jnp.max(xm, axis=0, keepdims=True)
        am = jnp.argmax(xm, axis=0, keepdims=True).astype(jnp.int32)
        pos = mx == jnp.float32(1.1)
        lab = jnp.where(pos, jnp.int32(1), jnp.int32(0))
        lab = jnp.where(ign, jnp.int32(-1), lab)
        mrows.append(jnp.where(hasany, am, 0))
        lrows.append(jnp.where(hasany, lab, 0))
    mt_ref[...] = jnp.concatenate(mrows, axis=0)
    lb_ref[...] = jnp.concatenate(lrows, axis=0)


def _assign(mod, ign3, onehot):
    return pl.pallas_call(
        _assign_body,
        grid=(NPAD // CB,),
        in_specs=[pl.BlockSpec((M, CB), lambda i: (0, i)),
                  pl.BlockSpec((1, 1, CB), lambda i: (i, 0, 0)),
                  pl.BlockSpec((M, NUM_CLASSES), lambda i: (0, 0))],
        out_specs=[pl.BlockSpec((NUM_CLASSES, CB), lambda i: (0, i)),
                   pl.BlockSpec((NUM_CLASSES, CB), lambda i: (0, i))],
        out_shape=[jax.ShapeDtypeStruct((NUM_CLASSES, NPAD), jnp.int32),
                   jax.ShapeDtypeStruct((NUM_CLASSES, NPAD), jnp.int32)],
    )(mod, ign3, onehot)


def kernel(centerness_matrix, targets, anchors, gt_labels):
    cvs, cis = [], []
    for lvl in range(5):
        start = LEVEL_STARTS[lvl]
        n_chunks, cw = LEVEL_CHUNKS[lvl]
        cv, ci = _level_topk(
            centerness_matrix[:, start:start + LEVEL_WIDTHS[lvl]],
            start, n_chunks, cw)
        cvs.append(cv[:, :K])
        cis.append(ci[:, :K])
    cand_val = jnp.concatenate(cvs, axis=1)
    cand_idx = jnp.concatenate(cis, axis=1)
    cand_val = jnp.pad(cand_val, ((0, 0), (0, CANDP - CAND)),
                       constant_values=-1.0)
    cand_idx = jnp.pad(cand_idx, ((0, 0), (0, CANDP - CAND)))

    mod_flat, ign_flat = _sc_scatter(
        centerness_matrix.reshape(-1),
        cand_idx.reshape(-1),
        cand_val.reshape(-1),
        targets.reshape(-1),
    )
    mod = mod_flat.reshape(M, NPAD)
    ign3 = ign_flat.reshape(NPAD // CB, 1, CB)

    onehot = (gt_labels[:, None] ==
              jnp.arange(NUM_CLASSES)[None, :]).astype(jnp.float32)
    mt, lb = _assign(mod, ign3, onehot)
    matches = mt[:, :N].T
    match_labels = lb[:, :N].T.astype(jnp.int8)
    return matches, match_labels
